# baseline (speedup 1.0000x reference)
import functools

import jax
import jax.numpy as jnp
from jax.experimental import pallas as pl
from jax.experimental.pallas import tpu as pltpu

_NUM_CLASSES = 11
_THRESHOLD = 0.5
_NEGPOS_RATIO = 3
_VAR0, _VAR1 = 0.1, 0.2
_CLASS_WEIGHT = [1.0, 2.925, 2.479, 1.448, 2.518, 1.719, 2.616, 3.526,
                 1.024, 1.832, 5.538]
_CHUNK = 64


def _mbl_kernel(loc_ref, conf_ref, pri_ref, tgt_ref, out_ref, *, num_priors):
    P = num_priors
    I = _CHUNK
    O = tgt_ref.shape[0]
    f32 = jnp.float32

    loc = loc_ref[...]
    conf = conf_ref[...]
    pri = pri_ref[...]
    tgt = tgt_ref[...]

    tx1 = tgt[:, :, 0:1]
    ty1 = tgt[:, :, 1:2]
    tx2 = tgt[:, :, 2:3]
    ty2 = tgt[:, :, 3:4]
    area_a = tgt[:, :, 4:5]

    px1, py1, px2, py2 = pri[0:1], pri[1:2], pri[2:3], pri[3:4]
    area_b = pri[4:5]

    ix = jnp.clip(jnp.minimum(tx2, px2) - jnp.maximum(tx1, px1), 0.0, None)
    iy = jnp.clip(jnp.minimum(ty2, py2) - jnp.maximum(ty1, py1), 0.0, None)
    inter = ix * iy
    ov = inter / (area_a + area_b - inter)

    lane = jax.lax.broadcasted_iota(jnp.int32, (1, 1, P), 2)
    sub_o = jax.lax.broadcasted_iota(jnp.int32, (O, 1, 1), 0)
    over_val = 2.0 + 0.001 * sub_o.astype(f32)

    rowmax = jnp.max(ov, axis=2, keepdims=True)
    bp = jnp.min(jnp.where(ov == rowmax, lane, P), axis=2, keepdims=True)

    ovf = jnp.where(bp == lane, over_val, ov)
    bto = jnp.max(ovf, axis=0, keepdims=True)
    posm = bto >= _THRESHOLD

    onehot_o = (ovf == bto).astype(f32)
    mt = jax.lax.dot_general(
        tgt[:, :, 5:10], onehot_o,
        dimension_numbers=(((0,), (0,)), ((1,), (1,))),
        preferred_element_type=f32)
    mcx = mt[:, 0:1, :].reshape(1, I, P)
    mcy = mt[:, 1:2, :].reshape(1, I, P)
    mlw5 = mt[:, 2:3, :].reshape(1, I, P)
    mlh5 = mt[:, 3:4, :].reshape(1, I, P)
    mlab1 = mt[:, 4:5, :].reshape(1, I, P)

    conf_t = jnp.where(posm, mlab1, 0.0)

    g_cx = (mcx - pri[5:6]) * pri[7:8]
    g_cy = (mcy - pri[6:7]) * pri[8:9]
    g_w = mlw5 - pri[9:10]
    g_h = mlh5 - pri[10:11]

    def sl1(d):
        ad = jnp.abs(d)
        return jnp.where(ad < 1.0, 0.5 * d * d, ad - 0.5)

    sl1sum = (sl1(loc[0:1] - g_cx) + sl1(loc[1:2] - g_cy) +
              sl1(loc[2:3] - g_w) + sl1(loc[3:4] - g_h))

    cw = jnp.full((1, I, P), _CLASS_WEIGHT[0], f32)
    gathered = conf[0:1]
    for c in range(1, _NUM_CLASSES):
        is_c = conf_t == float(c)
        cw = jnp.where(is_c, _CLASS_WEIGHT[c], cw)
        gathered = jnp.where(is_c, conf[c:c + 1], gathered)

    w_loc = jnp.where(posm, cw, 0.0)
    loss_l = jnp.sum(w_loc * sl1sum, axis=2, keepdims=True)

    s = jnp.sum(jnp.exp(conf), axis=0, keepdims=True)
    lse = jnp.log(s)
    lc = lse - gathered

    npos = jnp.sum(posm.astype(jnp.int32), axis=2, keepdims=True)
    mined = jnp.where(posm, 0.0, lc)
    bits = jax.lax.bitcast_convert_type(mined, jnp.int32)
    k = jnp.minimum(_NEGPOS_RATIO * npos, P - 1)

    def body(_, lohi):
        lo, hi = lohi
        mid = lo + (hi - lo) // 2
        cnt = jnp.sum((bits >= mid).astype(jnp.int32), axis=2, keepdims=True)
        ge = cnt >= k
        return jnp.where(ge, mid, lo), jnp.where(ge, hi, mid)

    init = (jnp.zeros((1, I, 1), jnp.int32),
            jnp.full((1, I, 1), 0x7F800000, jnp.int32))
    lo, _ = jax.lax.fori_loop(0, 31, body, init)
    sel = jnp.logical_or(posm, bits >= lo)
    loss_c = jnp.sum(jnp.where(sel, cw * lc, 0.0), axis=2, keepdims=True)

    npf = npos.astype(f32)
    out_ref[...] = jnp.concatenate(
        [loss_l[0], loss_c[0], npf[0], jnp.zeros((I, 1), f32)],
        axis=1)[None]


def kernel(loc_data, conf_data, priors, targets):
    B, P, _ = loc_data.shape
    O = targets.shape[1]
    nchunk = B // _CHUNK

    loc_t = jnp.transpose(loc_data, (2, 0, 1))
    conf_t = jnp.transpose(conf_data, (2, 0, 1))

    pcx, pcy = priors[:, 0], priors[:, 1]
    pw, ph = priors[:, 2], priors[:, 3]
    pri_aug = jnp.stack([
        pcx - pw * 0.5, pcy - ph * 0.5, pcx + pw * 0.5, pcy + ph * 0.5,
        pw * ph, pcx, pcy,
        1.0 / (_VAR0 * pw), 1.0 / (_VAR0 * ph),
        jnp.log(pw) / _VAR1, jnp.log(ph) / _VAR1,
    ], axis=0).reshape(11, 1, P)

    tx1, ty1 = targets[:, :, 0], targets[:, :, 1]
    tx2, ty2 = targets[:, :, 2], targets[:, :, 3]
    lab = targets[:, :, 4]
    tgt_aug = jnp.stack([
        tx1, ty1, tx2, ty2,
        (tx2 - tx1) * (ty2 - ty1),
        (tx1 + tx2) * 0.5, (ty1 + ty2) * 0.5,
        jnp.log(tx2 - tx1) / _VAR1, jnp.log(ty2 - ty1) / _VAR1,
        lab + 1.0,
    ], axis=2)
    tgt_t = jnp.transpose(tgt_aug, (1, 0, 2))

    out = pl.pallas_call(
        functools.partial(_mbl_kernel, num_priors=P),
        grid=(nchunk,),
        in_specs=[
            pl.BlockSpec((4, _CHUNK, P), lambda b: (0, b, 0)),
            pl.BlockSpec((_NUM_CLASSES, _CHUNK, P), lambda b: (0, b, 0)),
            pl.BlockSpec((11, 1, P), lambda b: (0, 0, 0)),
            pl.BlockSpec((O, _CHUNK, 10), lambda b: (0, b, 0)),
        ],
        out_specs=pl.BlockSpec((1, _CHUNK, 4), lambda b: (b, 0, 0)),
        out_shape=jax.ShapeDtypeStruct((nchunk, _CHUNK, 4), jnp.float32),
        compiler_params=pltpu.CompilerParams(
            dimension_semantics=("parallel",)),
    )(loc_t, conf_t, pri_aug, tgt_t)

    loss_l = jnp.sum(out[:, :, 0])
    loss_c = jnp.sum(out[:, :, 1])
    n = jnp.maximum(jnp.sum(out[:, :, 2]), 1.0)
    return loss_l / n, loss_c / n


# speedup vs baseline: 1.0235x; 1.0235x over previous
import functools

import jax
import jax.numpy as jnp
from jax.experimental import pallas as pl
from jax.experimental.pallas import tpu as pltpu

_NUM_CLASSES = 11
_THRESHOLD = 0.5
_NEGPOS_RATIO = 3
_VAR0, _VAR1 = 0.1, 0.2
_CLASS_WEIGHT = [1.0, 2.925, 2.479, 1.448, 2.518, 1.719, 2.616, 3.526,
                 1.024, 1.832, 5.538]
_CHUNK = 32


def _mbl_kernel(loc_ref, conf_ref, pri_ref, tgt_ref, out_ref, *, num_priors):
    P = num_priors
    I = _CHUNK
    O = tgt_ref.shape[0]
    f32 = jnp.float32

    loc = loc_ref[...]
    conf = conf_ref[...]
    pri = pri_ref[...]
    tgt = tgt_ref[...]

    tx1 = tgt[:, :, 0:1]
    ty1 = tgt[:, :, 1:2]
    tx2 = tgt[:, :, 2:3]
    ty2 = tgt[:, :, 3:4]
    area_a = tgt[:, :, 4:5]

    px1, py1, px2, py2 = pri[0:1], pri[1:2], pri[2:3], pri[3:4]
    area_b = pri[4:5]

    ix = jnp.clip(jnp.minimum(tx2, px2) - jnp.maximum(tx1, px1), 0.0, None)
    iy = jnp.clip(jnp.minimum(ty2, py2) - jnp.maximum(ty1, py1), 0.0, None)
    inter = ix * iy
    ov = inter / (area_a + area_b - inter)

    lane = jax.lax.broadcasted_iota(jnp.int32, (1, 1, P), 2)
    sub_o = jax.lax.broadcasted_iota(jnp.int32, (O, 1, 1), 0)
    over_val = 2.0 + 0.001 * sub_o.astype(f32)

    rowmax = jnp.max(ov, axis=2, keepdims=True)
    bp = jnp.min(jnp.where(ov == rowmax, lane, P), axis=2, keepdims=True)

    ovf = jnp.where(bp == lane, over_val, ov)
    bto = jnp.max(ovf, axis=0, keepdims=True)
    posm = bto >= _THRESHOLD

    onehot_o = (ovf == bto).astype(f32)
    mt = jax.lax.dot_general(
        tgt[:, :, 5:10], onehot_o,
        dimension_numbers=(((0,), (0,)), ((1,), (1,))),
        preferred_element_type=f32)
    mcx = mt[:, 0:1, :].reshape(1, I, P)
    mcy = mt[:, 1:2, :].reshape(1, I, P)
    mlw5 = mt[:, 2:3, :].reshape(1, I, P)
    mlh5 = mt[:, 3:4, :].reshape(1, I, P)
    mlab1 = mt[:, 4:5, :].reshape(1, I, P)

    conf_t = jnp.where(posm, mlab1, 0.0)

    g_cx = (mcx - pri[5:6]) * pri[7:8]
    g_cy = (mcy - pri[6:7]) * pri[8:9]
    g_w = mlw5 - pri[9:10]
    g_h = mlh5 - pri[10:11]

    def sl1(d):
        ad = jnp.abs(d)
        return jnp.where(ad < 1.0, 0.5 * d * d, ad - 0.5)

    sl1sum = (sl1(loc[0:1] - g_cx) + sl1(loc[1:2] - g_cy) +
              sl1(loc[2:3] - g_w) + sl1(loc[3:4] - g_h))

    cw = jnp.full((1, I, P), _CLASS_WEIGHT[0], f32)
    gathered = conf[0:1]
    for c in range(1, _NUM_CLASSES):
        is_c = conf_t == float(c)
        cw = jnp.where(is_c, _CLASS_WEIGHT[c], cw)
        gathered = jnp.where(is_c, conf[c:c + 1], gathered)

    w_loc = jnp.where(posm, cw, 0.0)
    loss_l = jnp.sum(w_loc * sl1sum, axis=2, keepdims=True)

    s = jnp.sum(jnp.exp(conf), axis=0, keepdims=True)
    lse = jnp.log(s)
    lc = lse - gathered

    npos = jnp.sum(posm.astype(jnp.int32), axis=2, keepdims=True)
    mined = jnp.where(posm, 0.0, lc)
    bits = jax.lax.bitcast_convert_type(mined, jnp.int32)
    k = jnp.minimum(_NEGPOS_RATIO * npos, P - 1)

    def body(_, lohi):
        lo, hi = lohi
        mid = lo + (hi - lo) // 2
        cnt = jnp.sum((bits >= mid).astype(jnp.int32), axis=2, keepdims=True)
        ge = cnt >= k
        return jnp.where(ge, mid, lo), jnp.where(ge, hi, mid)

    init = (jnp.zeros((1, I, 1), jnp.int32),
            jnp.full((1, I, 1), 0x7F800000, jnp.int32))
    lo, _ = jax.lax.fori_loop(0, 31, body, init)
    sel = jnp.logical_or(posm, bits >= lo)
    loss_c = jnp.sum(jnp.where(sel, cw * lc, 0.0), axis=2, keepdims=True)

    npf = npos.astype(f32)
    out_ref[...] = jnp.concatenate(
        [loss_l[0], loss_c[0], npf[0], jnp.zeros((I, 1), f32)],
        axis=1)[None]


def kernel(loc_data, conf_data, priors, targets):
    B, P, _ = loc_data.shape
    O = targets.shape[1]
    nchunk = B // _CHUNK

    loc_t = jnp.transpose(loc_data, (2, 0, 1))
    conf_t = jnp.transpose(conf_data, (2, 0, 1))

    pcx, pcy = priors[:, 0], priors[:, 1]
    pw, ph = priors[:, 2], priors[:, 3]
    pri_aug = jnp.stack([
        pcx - pw * 0.5, pcy - ph * 0.5, pcx + pw * 0.5, pcy + ph * 0.5,
        pw * ph, pcx, pcy,
        1.0 / (_VAR0 * pw), 1.0 / (_VAR0 * ph),
        jnp.log(pw) / _VAR1, jnp.log(ph) / _VAR1,
    ], axis=0).reshape(11, 1, P)

    tx1, ty1 = targets[:, :, 0], targets[:, :, 1]
    tx2, ty2 = targets[:, :, 2], targets[:, :, 3]
    lab = targets[:, :, 4]
    tgt_aug = jnp.stack([
        tx1, ty1, tx2, ty2,
        (tx2 - tx1) * (ty2 - ty1),
        (tx1 + tx2) * 0.5, (ty1 + ty2) * 0.5,
        jnp.log(tx2 - tx1) / _VAR1, jnp.log(ty2 - ty1) / _VAR1,
        lab + 1.0,
    ], axis=2)
    tgt_t = jnp.transpose(tgt_aug, (1, 0, 2))

    out = pl.pallas_call(
        functools.partial(_mbl_kernel, num_priors=P),
        grid=(nchunk,),
        in_specs=[
            pl.BlockSpec((4, _CHUNK, P), lambda b: (0, b, 0)),
            pl.BlockSpec((_NUM_CLASSES, _CHUNK, P), lambda b: (0, b, 0)),
            pl.BlockSpec((11, 1, P), lambda b: (0, 0, 0)),
            pl.BlockSpec((O, _CHUNK, 10), lambda b: (0, b, 0)),
        ],
        out_specs=pl.BlockSpec((1, _CHUNK, 4), lambda b: (b, 0, 0)),
        out_shape=jax.ShapeDtypeStruct((nchunk, _CHUNK, 4), jnp.float32),
        compiler_params=pltpu.CompilerParams(
            dimension_semantics=("parallel",)),
    )(loc_t, conf_t, pri_aug, tgt_t)

    loss_l = jnp.sum(out[:, :, 0])
    loss_c = jnp.sum(out[:, :, 1])
    n = jnp.maximum(jnp.sum(out[:, :, 2]), 1.0)
    return loss_l / n, loss_c / n


# speedup vs baseline: 1.0254x; 1.0018x over previous
import functools

import jax
import jax.numpy as jnp
from jax.experimental import pallas as pl
from jax.experimental.pallas import tpu as pltpu

_NUM_CLASSES = 11
_THRESHOLD = 0.5
_NEGPOS_RATIO = 3
_VAR0, _VAR1 = 0.1, 0.2
_CLASS_WEIGHT = [1.0, 2.925, 2.479, 1.448, 2.518, 1.719, 2.616, 3.526,
                 1.024, 1.832, 5.538]
_CHUNK = 32


def _mbl_kernel(loc_ref, conf_ref, pri_ref, tgt_ref, out_ref, *, num_priors):
    P = num_priors
    I = _CHUNK
    O = tgt_ref.shape[0]
    f32 = jnp.float32

    loc = loc_ref[...]
    conf = conf_ref[...]
    pri = pri_ref[...]
    tgt = tgt_ref[...]

    tx1 = tgt[:, :, 0:1]
    ty1 = tgt[:, :, 1:2]
    tx2 = tgt[:, :, 2:3]
    ty2 = tgt[:, :, 3:4]
    area_a = tgt[:, :, 4:5]

    px1, py1, px2, py2 = pri[0:1], pri[1:2], pri[2:3], pri[3:4]
    area_b = pri[4:5]

    ix = jnp.clip(jnp.minimum(tx2, px2) - jnp.maximum(tx1, px1), 0.0, None)
    iy = jnp.clip(jnp.minimum(ty2, py2) - jnp.maximum(ty1, py1), 0.0, None)
    inter = ix * iy
    ov = inter / (area_a + area_b - inter)

    lane = jax.lax.broadcasted_iota(jnp.int32, (1, 1, P), 2)
    sub_o = jax.lax.broadcasted_iota(jnp.int32, (O, 1, 1), 0)
    over_val = 2.0 + 0.001 * sub_o.astype(f32)

    rowmax = jnp.max(ov, axis=2, keepdims=True)
    bp = jnp.min(jnp.where(ov == rowmax, lane, P), axis=2, keepdims=True)

    ovf = jnp.where(bp == lane, over_val, ov)
    bto = jnp.max(ovf, axis=0, keepdims=True)
    posm = bto >= _THRESHOLD

    onehot_o = (ovf == bto).astype(f32)
    mt = jax.lax.dot_general(
        tgt[:, :, 5:10], onehot_o,
        dimension_numbers=(((0,), (0,)), ((1,), (1,))),
        preferred_element_type=f32)
    mcx = mt[:, 0:1, :].reshape(1, I, P)
    mcy = mt[:, 1:2, :].reshape(1, I, P)
    mlw5 = mt[:, 2:3, :].reshape(1, I, P)
    mlh5 = mt[:, 3:4, :].reshape(1, I, P)
    mlab1 = mt[:, 4:5, :].reshape(1, I, P)

    conf_t = jnp.where(posm, mlab1, 0.0)

    g_cx = (mcx - pri[5:6]) * pri[7:8]
    g_cy = (mcy - pri[6:7]) * pri[8:9]
    g_w = mlw5 - pri[9:10]
    g_h = mlh5 - pri[10:11]

    def sl1(d):
        ad = jnp.abs(d)
        return jnp.where(ad < 1.0, 0.5 * d * d, ad - 0.5)

    sl1sum = (sl1(loc[0:1] - g_cx) + sl1(loc[1:2] - g_cy) +
              sl1(loc[2:3] - g_w) + sl1(loc[3:4] - g_h))

    cw = jnp.full((1, I, P), _CLASS_WEIGHT[0], f32)
    gathered = conf[0:1]
    for c in range(1, _NUM_CLASSES):
        is_c = conf_t == float(c)
        cw = jnp.where(is_c, _CLASS_WEIGHT[c], cw)
        gathered = jnp.where(is_c, conf[c:c + 1], gathered)

    w_loc = jnp.where(posm, cw, 0.0)
    loss_l = jnp.sum(w_loc * sl1sum, axis=2, keepdims=True)

    s = jnp.sum(jnp.exp(conf), axis=0, keepdims=True)
    lse = jnp.log(s)
    lc = lse - gathered

    npos = jnp.sum(posm.astype(jnp.int32), axis=2, keepdims=True)
    mined = jnp.where(posm, 0.0, lc)
    bits = jax.lax.bitcast_convert_type(mined, jnp.int32)
    k = jnp.minimum(_NEGPOS_RATIO * npos, P - 1)

    def body(_, lohi):
        lo, hi = lohi
        mid = lo + (hi - lo) // 2
        cnt = jnp.sum((bits >= mid).astype(jnp.int32), axis=2, keepdims=True)
        ge = cnt >= k
        return jnp.where(ge, mid, lo), jnp.where(ge, hi, mid)

    init = (jnp.zeros((1, I, 1), jnp.int32),
            jnp.full((1, I, 1), 0x7F800000, jnp.int32))
    lo, _ = jax.lax.fori_loop(0, 31, body, init)
    sel = jnp.logical_or(posm, bits >= lo)
    loss_c = jnp.sum(jnp.where(sel, cw * lc, 0.0), axis=2, keepdims=True)

    npf = npos.astype(f32)
    out_ref[...] = jnp.concatenate(
        [loss_l[0], loss_c[0], npf[0], jnp.zeros((I, 1), f32)],
        axis=1)[None]


def kernel(loc_data, conf_data, priors, targets):
    B, P, _ = loc_data.shape
    O = targets.shape[1]
    nchunk = B // _CHUNK

    loc_t = jnp.transpose(loc_data, (2, 0, 1))
    conf_t = jnp.transpose(conf_data, (2, 0, 1))

    pcx, pcy = priors[:, 0], priors[:, 1]
    pw, ph = priors[:, 2], priors[:, 3]
    px1, py1 = pcx - pw * 0.5, pcy - ph * 0.5
    px2, py2 = pcx + pw * 0.5, pcy + ph * 0.5
    pri_aug = jnp.stack([
        px1, py1, px2, py2,
        (px2 - px1) * (py2 - py1), pcx, pcy,
        1.0 / (_VAR0 * pw), 1.0 / (_VAR0 * ph),
        jnp.log(pw) / _VAR1, jnp.log(ph) / _VAR1,
    ], axis=0).reshape(11, 1, P)

    tx1, ty1 = targets[:, :, 0], targets[:, :, 1]
    tx2, ty2 = targets[:, :, 2], targets[:, :, 3]
    lab = targets[:, :, 4]
    tgt_aug = jnp.stack([
        tx1, ty1, tx2, ty2,
        (tx2 - tx1) * (ty2 - ty1),
        (tx1 + tx2) * 0.5, (ty1 + ty2) * 0.5,
        jnp.log(tx2 - tx1) / _VAR1, jnp.log(ty2 - ty1) / _VAR1,
        lab + 1.0,
    ], axis=2)
    tgt_t = jnp.transpose(tgt_aug, (1, 0, 2))

    out = pl.pallas_call(
        functools.partial(_mbl_kernel, num_priors=P),
        grid=(nchunk,),
        in_specs=[
            pl.BlockSpec((4, _CHUNK, P), lambda b: (0, b, 0)),
            pl.BlockSpec((_NUM_CLASSES, _CHUNK, P), lambda b: (0, b, 0)),
            pl.BlockSpec((11, 1, P), lambda b: (0, 0, 0)),
            pl.BlockSpec((O, _CHUNK, 10), lambda b: (0, b, 0)),
        ],
        out_specs=pl.BlockSpec((1, _CHUNK, 4), lambda b: (b, 0, 0)),
        out_shape=jax.ShapeDtypeStruct((nchunk, _CHUNK, 4), jnp.float32),
        compiler_params=pltpu.CompilerParams(
            dimension_semantics=("parallel",)),
    )(loc_t, conf_t, pri_aug, tgt_t)

    loss_l = jnp.sum(out[:, :, 0])
    loss_c = jnp.sum(out[:, :, 1])
    n = jnp.maximum(jnp.sum(out[:, :, 2]), 1.0)
    return loss_l / n, loss_c / n
